# baseline (speedup 1.0000x reference)
import jax
import jax.numpy as jnp
from jax.experimental import pallas as pl

_N_LAYERS = 3
_PER_LAYER_OPS = 19


def _fused_body(*refs):
    out_ref = refs[-1]
    vals = [r[...] for r in refs[:-1]]
    it = iter(vals)
    x = next(it)
    projWt = next(it)
    proj_b = next(it)
    mem = next(it)

    h = jnp.dot(x, projWt, preferred_element_type=jnp.float32) + proj_b
    for _ in range(_N_LAYERS):
        (W_ir, W_iz, W_in, W_hr, W_hz, W_hn,
         b_ir, b_iz, b_in, b_hr, b_hz, b_hn,
         W1h, W1m, b1, W2, b2, Wa, ba) = [next(it) for _ in range(_PER_LAYER_OPS)]

        xm = jnp.mean(h, axis=0, keepdims=True)
        i_r = jnp.dot(xm, W_ir, preferred_element_type=jnp.float32) + b_ir
        i_z = jnp.dot(xm, W_iz, preferred_element_type=jnp.float32) + b_iz
        i_n = jnp.dot(xm, W_in, preferred_element_type=jnp.float32) + b_in
        h_r = jnp.dot(mem, W_hr, preferred_element_type=jnp.float32) + b_hr
        h_z = jnp.dot(mem, W_hz, preferred_element_type=jnp.float32) + b_hz
        h_n = jnp.dot(mem, W_hn, preferred_element_type=jnp.float32) + b_hn
        r = jax.nn.sigmoid(i_r + h_r)
        z = jax.nn.sigmoid(i_z + h_z)
        n = jnp.tanh(i_n + r * h_n)
        mem = (1.0 - z) * n + z * mem

        mvec = jnp.dot(mem, W1m, preferred_element_type=jnp.float32) + b1
        m1 = jax.nn.relu(jnp.dot(h, W1h, preferred_element_type=jnp.float32) + mvec)
        msgs = jnp.dot(m1, W2, preferred_element_type=jnp.float32) + b2
        h = jnp.dot(msgs, Wa, preferred_element_type=jnp.float32) + ba

    cW1 = next(it)
    cb1 = next(it)
    cW2 = next(it)
    cb2 = next(it)
    c1 = jax.nn.relu(jnp.dot(h, cW1, preferred_element_type=jnp.float32) + cb1)
    out_ref[...] = jnp.dot(c1, cW2, preferred_element_type=jnp.float32) + cb2


def kernel(x, edge_index, edge_attr, edge_time, params):
    p = params
    d_mem = p['memory'].shape[1]
    operands = [x, p['proj_W'].T, p['proj_b'].reshape(1, -1), p['memory']]
    for lp in p['layers']:
        Wih_t = lp['Wih'].T
        Whh_t = lp['Whh'].T
        bih = lp['bih'].reshape(1, -1)
        bhh = lp['bhh'].reshape(1, -1)
        operands += [
            Wih_t[:, 0:d_mem], Wih_t[:, d_mem:2 * d_mem], Wih_t[:, 2 * d_mem:],
            Whh_t[:, 0:d_mem], Whh_t[:, d_mem:2 * d_mem], Whh_t[:, 2 * d_mem:],
            bih[:, 0:d_mem], bih[:, d_mem:2 * d_mem], bih[:, 2 * d_mem:],
            bhh[:, 0:d_mem], bhh[:, d_mem:2 * d_mem], bhh[:, 2 * d_mem:],
            lp['msg_W1'][:, :p['proj_W'].shape[0]].T,
            lp['msg_W1'][:, p['proj_W'].shape[0]:].T,
            lp['msg_b1'].reshape(1, -1),
            lp['msg_W2'].T, lp['msg_b2'].reshape(1, -1),
            lp['agg_W'].T, lp['agg_b'].reshape(1, -1),
        ]
    operands += [p['cls_W1'].T, p['cls_b1'].reshape(1, -1),
                 p['cls_W2'].T, p['cls_b2'].reshape(1, -1)]

    return pl.pallas_call(
        _fused_body,
        out_shape=jax.ShapeDtypeStruct((x.shape[0], 1), jnp.float32),
    )(*operands)


# speedup vs baseline: 1.0699x; 1.0699x over previous
import jax
import jax.numpy as jnp
from jax.experimental import pallas as pl

_N_LAYERS = 3
_PER_LAYER_OPS = 19


def _dot(a, b):
    return jnp.dot(a, b, preferred_element_type=jnp.float32)


def _fused_body(*refs):
    out_ref = refs[-1]
    vals = [r[...] for r in refs[:-1]]
    it = iter(vals)
    x = next(it)
    projWt = next(it)
    proj_b = next(it)
    mem = next(it)
    layers = [[next(it) for _ in range(_PER_LAYER_OPS)] for _ in range(_N_LAYERS)]
    cW1 = next(it)
    cb1 = next(it)
    cW2 = next(it)
    cb2 = next(it)

    a = x
    M = projWt
    c = proj_b
    hbar = _dot(jnp.mean(x, axis=0, keepdims=True), projWt) + proj_b
    for l in range(_N_LAYERS):
        (W_ir, W_iz, W_in, W_hr, W_hz, W_hn,
         b_ir, b_iz, b_in, b_hr, b_hz, b_hn,
         W1h, W1m, b1, W2, b2, Wa, ba) = layers[l]

        r = jax.nn.sigmoid(_dot(hbar, W_ir) + b_ir + _dot(mem, W_hr) + b_hr)
        z = jax.nn.sigmoid(_dot(hbar, W_iz) + b_iz + _dot(mem, W_hz) + b_hz)
        n = jnp.tanh(_dot(hbar, W_in) + b_in + r * (_dot(mem, W_hn) + b_hn))
        mem = (1.0 - z) * n + z * mem

        mvec = _dot(mem, W1m) + b1
        F = _dot(M, W1h)
        g = _dot(c, W1h) + mvec
        a = jax.nn.relu(_dot(a, F) + g)
        M = _dot(W2, Wa)
        c = _dot(b2, Wa) + ba
        if l + 1 < _N_LAYERS:
            hbar = _dot(jnp.mean(a, axis=0, keepdims=True), M) + c

    Fc = _dot(M, cW1)
    gc = _dot(c, cW1) + cb1
    c1 = jax.nn.relu(_dot(a, Fc) + gc)
    out_ref[...] = _dot(c1, cW2) + cb2


def kernel(x, edge_index, edge_attr, edge_time, params):
    p = params
    d_mem = p['memory'].shape[1]
    operands = [x, p['proj_W'].T, p['proj_b'].reshape(1, -1), p['memory']]
    for lp in p['layers']:
        Wih_t = lp['Wih'].T
        Whh_t = lp['Whh'].T
        bih = lp['bih'].reshape(1, -1)
        bhh = lp['bhh'].reshape(1, -1)
        operands += [
            Wih_t[:, 0:d_mem], Wih_t[:, d_mem:2 * d_mem], Wih_t[:, 2 * d_mem:],
            Whh_t[:, 0:d_mem], Whh_t[:, d_mem:2 * d_mem], Whh_t[:, 2 * d_mem:],
            bih[:, 0:d_mem], bih[:, d_mem:2 * d_mem], bih[:, 2 * d_mem:],
            bhh[:, 0:d_mem], bhh[:, d_mem:2 * d_mem], bhh[:, 2 * d_mem:],
            lp['msg_W1'][:, :p['proj_W'].shape[0]].T,
            lp['msg_W1'][:, p['proj_W'].shape[0]:].T,
            lp['msg_b1'].reshape(1, -1),
            lp['msg_W2'].T, lp['msg_b2'].reshape(1, -1),
            lp['agg_W'].T, lp['agg_b'].reshape(1, -1),
        ]
    operands += [p['cls_W1'].T, p['cls_b1'].reshape(1, -1),
                 p['cls_W2'].T, p['cls_b2'].reshape(1, -1)]

    return pl.pallas_call(
        _fused_body,
        out_shape=jax.ShapeDtypeStruct((x.shape[0], 1), jnp.float32),
    )(*operands)


# speedup vs baseline: 1.8027x; 1.6850x over previous
import jax
import jax.numpy as jnp
from jax.experimental import pallas as pl

_N_LAYERS = 3
_PER_LAYER_OPS = 10


def _dot(a, b):
    return jax.lax.dot_general(a, b, (((1,), (0,)), ((), ())),
                               preferred_element_type=jnp.float32)


def _dot_t(a, b):
    return jax.lax.dot_general(a, b, (((1,), (1,)), ((), ())),
                               preferred_element_type=jnp.float32)


def _fused_body(*refs):
    out_ref = refs[-1]
    vals = [r[...] for r in refs[:-1]]
    it = iter(vals)
    x = next(it)
    proj_W = next(it)
    proj_b = next(it)
    mem = next(it)
    layers = [[next(it) for _ in range(_PER_LAYER_OPS)] for _ in range(_N_LAYERS)]
    cls_W1 = next(it)
    cls_b1 = next(it)
    cls_W2 = next(it)

    d_h = proj_W.shape[0]
    d_mem = mem.shape[1]

    a = x
    Mt = proj_W
    c = proj_b
    hbar = _dot_t(jnp.mean(x, axis=0, keepdims=True), Mt) + c
    for l in range(_N_LAYERS):
        (Wih, bih, Whh, bhh, msg_W1, msg_b1,
         msg_W2, msg_b2, agg_W, agg_b) = layers[l]

        gi = _dot_t(hbar, Wih) + bih
        gh = _dot_t(mem, Whh) + bhh
        r = jax.nn.sigmoid(gi[:, 0:d_mem] + gh[:, 0:d_mem])
        z = jax.nn.sigmoid(gi[:, d_mem:2 * d_mem] + gh[:, d_mem:2 * d_mem])
        n = jnp.tanh(gi[:, 2 * d_mem:] + r * gh[:, 2 * d_mem:])
        mem = (1.0 - z) * n + z * mem

        W1h = msg_W1[:, :d_h]
        mvec = _dot_t(mem, msg_W1[:, d_h:]) + msg_b1
        G = _dot(W1h, Mt)
        g = _dot_t(c, W1h) + mvec
        a = jax.nn.relu(_dot_t(a, G) + g)
        Mt = _dot(agg_W, msg_W2)
        c = _dot_t(msg_b2, agg_W) + agg_b
        if l + 1 < _N_LAYERS:
            hbar = _dot_t(jnp.mean(a, axis=0, keepdims=True), Mt) + c

    Gc = _dot(cls_W1, Mt)
    gc = _dot_t(c, cls_W1) + cls_b1
    c1 = jax.nn.relu(_dot_t(a, Gc) + gc)
    out_ref[...] = _dot_t(c1, cls_W2)


def kernel(x, edge_index, edge_attr, edge_time, params):
    p = params
    operands = [x, p['proj_W'], p['proj_b'].reshape(1, -1), p['memory']]
    for lp in p['layers']:
        operands += [lp['Wih'], lp['bih'].reshape(1, -1),
                     lp['Whh'], lp['bhh'].reshape(1, -1),
                     lp['msg_W1'], lp['msg_b1'].reshape(1, -1),
                     lp['msg_W2'], lp['msg_b2'].reshape(1, -1),
                     lp['agg_W'], lp['agg_b'].reshape(1, -1)]
    operands += [p['cls_W1'], p['cls_b1'].reshape(1, -1), p['cls_W2']]

    out = pl.pallas_call(
        _fused_body,
        out_shape=jax.ShapeDtypeStruct((x.shape[0], 1), jnp.float32),
    )(*operands)
    return out + p['cls_b2']
